# baseline (speedup 1.0000x reference)
import functools

import jax
import jax.numpy as jnp
from jax import lax
from jax.experimental import pallas as pl
from jax.experimental.pallas import tpu as pltpu
from jax.experimental.pallas import tpu_sc as plsc

_L = 16
_NC = 2
_NS = 16


@functools.lru_cache(maxsize=None)
def _make_sc_argmax(B, S, V, NC):
    NW = NC * _NS
    RPW = B // NW
    BPC = B // NC
    CH = V // _L
    G = _L // RPW
    NCOL = BPC // _L
    mesh = plsc.VectorSubcoreMesh(core_axis_name="c", subcore_axis_name="s",
                                  num_cores=NC)

    @functools.partial(
        pl.kernel,
        out_type=jax.ShapeDtypeStruct((B,), jnp.int32),
        mesh=mesh,
        scratch_types=[
            pltpu.VMEM((RPW, V), jnp.float32),
            pltpu.VMEM((_L,), jnp.int32),
            pltpu.VMEM_SHARED((_NS * _L,), jnp.int32),
            pltpu.VMEM((G * _L,), jnp.int32),
            pltpu.VMEM((_L,), jnp.int32),
            pltpu.SemaphoreType.DMA((RPW,)),
        ],
    )
    def k(probas_hbm, cand_hbm, rows_v, cand_v, shared_c, quad_v, out_c,
          in_sems):
        cid = lax.axis_index("c")
        sid = lax.axis_index("s")
        wid = cid * _NS + sid
        base = wid * RPW
        loads = [
            pltpu.async_copy(probas_hbm.at[base + r, S - 1], rows_v.at[r],
                             in_sems.at[r])
            for r in range(RPW)
        ]
        lane = lax.iota(jnp.int32, _L)
        cand_vec = jnp.zeros((_L,), jnp.int32)
        SETS = 4
        for r in range(RPW):
            loads[r].wait()

            def body(i, carry, r=r):
                out = []
                for j in range(SETS):
                    maxv, maxi, idx = carry[j]
                    v = rows_v[r, pl.ds((i * SETS + j) * _L, _L)]
                    upd = v > maxv
                    out.append((
                        jnp.where(upd, v, maxv),
                        jnp.where(upd, idx, maxi),
                        idx + SETS * _L,
                    ))
                return tuple(out)

            init = tuple(
                (jnp.full((_L,), -jnp.inf, jnp.float32),
                 jnp.zeros((_L,), jnp.int32),
                 lane + j * _L)
                for j in range(SETS))
            acc = lax.fori_loop(0, CH // SETS, body, init, unroll=1)
            maxv, maxi, _ = acc[0]
            for j in range(1, SETS):
                ov, oi, _ = acc[j]
                upd = (ov > maxv) | ((ov == maxv) & (oi < maxi))
                maxv = jnp.where(upd, ov, maxv)
                maxi = jnp.where(upd, oi, maxi)
            for s in (1, 2, 4, 8):
                perm = jnp.bitwise_xor(lane, s)
                ov = maxv.at[perm].get(mode="promise_in_bounds")
                oi = maxi.at[perm].get(mode="promise_in_bounds")
                upd = (ov > maxv) | ((ov == maxv) & (oi < maxi))
                maxv = jnp.where(upd, ov, maxv)
                maxi = jnp.where(upd, oi, maxi)
            cand_vec = jnp.where(lane == r, maxi, cand_vec)
        cand_v[...] = cand_vec
        pltpu.sync_copy(cand_v, shared_c.at[pl.ds(sid * _L, _L)])
        plsc.subcore_barrier()

        t = sid & (NCOL - 1)
        pltpu.sync_copy(shared_c.at[pl.ds(t * G * _L, G * _L)], quad_v)
        parts = [quad_v[pl.ds(j * _L, _L)] for j in range(G)]
        merged = parts[0]
        for j in range(1, G):
            shuf = parts[j].at[(lane - j * RPW) & (_L - 1)].get(
                mode="promise_in_bounds")
            merged = jnp.where(lane < j * RPW, merged, shuf)
        out_c[...] = merged

        @pl.when(sid < NCOL)
        def _flush():
            pltpu.sync_copy(out_c, cand_hbm.at[pl.ds(cid * BPC + t * _L, _L)])

    return k


@functools.lru_cache(maxsize=None)
def _make_tc_slice_copy(B, S, V):
    CHUNK = 32
    NSLOT = 3
    LA = 2
    n = B // CHUNK

    def body(x_hbm, o_hbm, buf, in_sems, out_sems):
        def start_in(k):
            b = k % NSLOT
            return pltpu.async_copy(
                x_hbm.at[pl.ds(k * CHUNK, CHUNK), S - 1], buf.at[b],
                in_sems.at[b])

        def start_out(k):
            b = k % NSLOT
            return pltpu.async_copy(
                buf.at[b], o_hbm.at[pl.ds(k * CHUNK, CHUNK)], out_sems.at[b])

        ins, outs = {}, {}
        for k in range(min(LA, n)):
            ins[k] = start_in(k)
        for k in range(n):
            ins[k].wait()
            outs[k] = start_out(k)
            nk = k + LA
            if nk < n:
                if nk >= NSLOT:
                    outs[nk - NSLOT].wait()
                ins[nk] = start_in(nk)
        for k in range(max(0, n - NSLOT), n):
            outs[k].wait()

    return pl.pallas_call(
        body,
        in_specs=[pl.BlockSpec(memory_space=pl.ANY)],
        out_specs=pl.BlockSpec(memory_space=pl.ANY),
        out_shape=jax.ShapeDtypeStruct((B, V), jnp.float32),
        scratch_shapes=[
            pltpu.VMEM((NSLOT, CHUNK, V), jnp.float32),
            pltpu.SemaphoreType.DMA((NSLOT,)),
            pltpu.SemaphoreType.DMA((NSLOT,)),
        ],
    )


def kernel(probas, greedy):
    B, S, V = probas.shape
    cand = _make_sc_argmax(B, S, V, _NC)(probas)
    dist = _make_tc_slice_copy(B, S, V)(probas)
    return (cand.reshape(B, 1), dist)


# speedup vs baseline: 1.0234x; 1.0234x over previous
import functools

import jax
import jax.numpy as jnp
from jax import lax
from jax.experimental import pallas as pl
from jax.experimental.pallas import tpu as pltpu
from jax.experimental.pallas import tpu_sc as plsc

_L = 16
_NC = 2
_NS = 16


@functools.lru_cache(maxsize=None)
def _make_sc_argmax(B, S, V, NC):
    NW = NC * _NS
    RPW = B // NW
    BPC = B // NC
    CH = V // _L
    G = _L // RPW
    NCOL = BPC // _L
    mesh = plsc.VectorSubcoreMesh(core_axis_name="c", subcore_axis_name="s",
                                  num_cores=NC)

    @functools.partial(
        pl.kernel,
        out_type=jax.ShapeDtypeStruct((B,), jnp.int32),
        mesh=mesh,
        scratch_types=[
            pltpu.VMEM((RPW, V), jnp.float32),
            pltpu.VMEM((_L,), jnp.int32),
            pltpu.VMEM_SHARED((_NS * _L,), jnp.int32),
            pltpu.VMEM((G * _L,), jnp.int32),
            pltpu.VMEM((_L,), jnp.int32),
            pltpu.SemaphoreType.DMA((RPW,)),
        ],
    )
    def k(probas_hbm, cand_hbm, rows_v, cand_v, shared_c, quad_v, out_c,
          in_sems):
        cid = lax.axis_index("c")
        sid = lax.axis_index("s")
        wid = cid * _NS + sid
        base = wid * RPW
        loads = [
            pltpu.async_copy(probas_hbm.at[base + r, S - 1], rows_v.at[r],
                             in_sems.at[r])
            for r in range(RPW)
        ]
        lane = lax.iota(jnp.int32, _L)
        cand_vec = jnp.zeros((_L,), jnp.int32)
        SETS = 4
        for r in range(RPW):
            loads[r].wait()

            def body(i, carry, r=r):
                out = []
                for j in range(SETS):
                    maxv, maxi, idx = carry[j]
                    v = rows_v[r, pl.ds((i * SETS + j) * _L, _L)]
                    upd = v > maxv
                    out.append((
                        jnp.where(upd, v, maxv),
                        jnp.where(upd, idx, maxi),
                        idx + SETS * _L,
                    ))
                return tuple(out)

            init = tuple(
                (jnp.full((_L,), -jnp.inf, jnp.float32),
                 jnp.zeros((_L,), jnp.int32),
                 lane + j * _L)
                for j in range(SETS))
            acc = lax.fori_loop(0, CH // SETS, body, init, unroll=2)
            maxv, maxi, _ = acc[0]
            for j in range(1, SETS):
                ov, oi, _ = acc[j]
                upd = (ov > maxv) | ((ov == maxv) & (oi < maxi))
                maxv = jnp.where(upd, ov, maxv)
                maxi = jnp.where(upd, oi, maxi)
            for s in (1, 2, 4, 8):
                perm = jnp.bitwise_xor(lane, s)
                ov = maxv.at[perm].get(mode="promise_in_bounds")
                oi = maxi.at[perm].get(mode="promise_in_bounds")
                upd = (ov > maxv) | ((ov == maxv) & (oi < maxi))
                maxv = jnp.where(upd, ov, maxv)
                maxi = jnp.where(upd, oi, maxi)
            cand_vec = jnp.where(lane == r, maxi, cand_vec)
        cand_v[...] = cand_vec
        pltpu.sync_copy(cand_v, shared_c.at[pl.ds(sid * _L, _L)])
        plsc.subcore_barrier()

        t = sid & (NCOL - 1)
        pltpu.sync_copy(shared_c.at[pl.ds(t * G * _L, G * _L)], quad_v)
        parts = [quad_v[pl.ds(j * _L, _L)] for j in range(G)]
        merged = parts[0]
        for j in range(1, G):
            shuf = parts[j].at[(lane - j * RPW) & (_L - 1)].get(
                mode="promise_in_bounds")
            merged = jnp.where(lane < j * RPW, merged, shuf)
        out_c[...] = merged

        @pl.when(sid < NCOL)
        def _flush():
            pltpu.sync_copy(out_c, cand_hbm.at[pl.ds(cid * BPC + t * _L, _L)])

    return k


@functools.lru_cache(maxsize=None)
def _make_tc_slice_copy(B, S, V):
    CHUNK = 32
    NSLOT = 3
    LA = 2
    n = B // CHUNK

    def body(x_hbm, o_hbm, buf, in_sems, out_sems):
        def start_in(k):
            b = k % NSLOT
            return pltpu.async_copy(
                x_hbm.at[pl.ds(k * CHUNK, CHUNK), S - 1], buf.at[b],
                in_sems.at[b])

        def start_out(k):
            b = k % NSLOT
            return pltpu.async_copy(
                buf.at[b], o_hbm.at[pl.ds(k * CHUNK, CHUNK)], out_sems.at[b])

        ins, outs = {}, {}
        for k in range(min(LA, n)):
            ins[k] = start_in(k)
        for k in range(n):
            ins[k].wait()
            outs[k] = start_out(k)
            nk = k + LA
            if nk < n:
                if nk >= NSLOT:
                    outs[nk - NSLOT].wait()
                ins[nk] = start_in(nk)
        for k in range(max(0, n - NSLOT), n):
            outs[k].wait()

    return pl.pallas_call(
        body,
        in_specs=[pl.BlockSpec(memory_space=pl.ANY)],
        out_specs=pl.BlockSpec(memory_space=pl.ANY),
        out_shape=jax.ShapeDtypeStruct((B, V), jnp.float32),
        scratch_shapes=[
            pltpu.VMEM((NSLOT, CHUNK, V), jnp.float32),
            pltpu.SemaphoreType.DMA((NSLOT,)),
            pltpu.SemaphoreType.DMA((NSLOT,)),
        ],
    )


def kernel(probas, greedy):
    B, S, V = probas.shape
    cand = _make_sc_argmax(B, S, V, _NC)(probas)
    dist = _make_tc_slice_copy(B, S, V)(probas)
    return (cand.reshape(B, 1), dist)
